# baseline (speedup 1.0000x reference)
import functools

import jax
import jax.numpy as jnp
from jax import lax
from jax.experimental import pallas as pl
from jax.experimental.pallas import tpu as pltpu
from jax.experimental.pallas import tpu_sc as plsc

D = 128
N_SEG = 200
N = 4096 * 200
NC, NS, L = 2, 16, 16
NW = NC * NS
ROWS_PER_W = N // NW
K = 128
N_CHUNKS = ROWS_PER_W // K
NBUF = 5


def _prep_body(seg_ref, st_ref, segtab_ref, sttab_ref, ctab_ref, cidx_ref):
    ctab_ref[0:N_SEG, :] = segtab_ref[...] + sttab_ref[0:1, :]
    ctab_ref[N_SEG:2 * N_SEG, :] = segtab_ref[...] + sttab_ref[1:2, :]
    cidx_ref[...] = st_ref[...] * N_SEG + seg_ref[...]


def _prep(segment, strand, segment_table, strand_table):
    return pl.pallas_call(
        _prep_body,
        out_shape=(
            jax.ShapeDtypeStruct((2 * N_SEG, D), jnp.float32),
            jax.ShapeDtypeStruct(segment.shape, jnp.int32),
        ),
    )(segment, strand, segment_table, strand_table)


def _sc_body(x_hbm, cidx_hbm, tab_hbm, ctab_hbm, out_hbm, csh, *rest):
    wid = lax.axis_index("s") * NC + lax.axis_index("c")
    row0 = wid * ROWS_PER_W
    xb = rest[0:NBUF]
    cb = rest[NBUF:2 * NBUF]
    toks = rest[2 * NBUF:3 * NBUF]
    semi = rest[3 * NBUF:4 * NBUF]
    semg = rest[4 * NBUF:5 * NBUF]
    sema = rest[5 * NBUF:6 * NBUF]
    semo = rest[6 * NBUF:7 * NBUF]

    @pl.when(lax.axis_index("s") == 0)
    def _stage_ctab():
        pltpu.sync_copy(ctab_hbm, csh)

    plsc.subcore_barrier()

    def idx_copies(g, b):
        sl = pl.ds(row0 + g * K, K)
        return (pltpu.make_async_copy(x_hbm.at[sl], xb[b], semi[b]),
                pltpu.make_async_copy(cidx_hbm.at[sl], cb[b], semi[b]))

    def issue_idx(g, b):
        for cp in idx_copies(g, b):
            cp.start()

    def tok_copy(b):
        return pltpu.make_async_copy(tab_hbm.at[xb[b]], toks[b], semg[b])

    def add_copy(b):
        return pltpu.make_async_copy(csh.at[cb[b]], toks[b], sema[b])

    def out_copy(g, b):
        return pltpu.make_async_copy(
            toks[b], out_hbm.at[pl.ds(row0 + g * K, K)], semo[b])

    def launch_tok(g, b):
        for cp in idx_copies(g, b):
            cp.wait()
        tok_copy(b).start()

    for j in range(min(NBUF - 1, 3)):
        issue_idx(j, j)
    launch_tok(0, 0)
    launch_tok(1, 1)

    def step(i, carry):
        for b in range(NBUF):
            g = NBUF * i + b

            def prev_out():
                add_copy((b - 1) % NBUF).wait()
                out_copy(g - 1, (b - 1) % NBUF).start()

            if b == 0:
                @pl.when(i >= 1)
                def _prev_out():
                    prev_out()
            else:
                prev_out()

            tok_copy(b).wait()
            add_copy(b).start(add=True)

            def next_tok():
                bn = (b + 2) % NBUF

                def drain():
                    out_copy(g - (NBUF - 2), bn).wait()

                if b < NBUF - 2:
                    @pl.when(i >= 1)
                    def _drain():
                        drain()
                else:
                    drain()
                launch_tok(g + 2, bn)

            if b <= NBUF - 3:
                next_tok()
            else:
                @pl.when(i <= N_CHUNKS // NBUF - 2)
                def _next_tok():
                    next_tok()

            if b <= NBUF - 4:
                issue_idx(g + 3, (b + 3) % NBUF)
            else:
                @pl.when(i <= N_CHUNKS // NBUF - 2)
                def _next_idx():
                    issue_idx(g + 3, (b + 3) % NBUF)
        return carry

    lax.fori_loop(0, N_CHUNKS // NBUF, step, 0)

    add_copy((N_CHUNKS - 1) % NBUF).wait()
    out_copy(N_CHUNKS - 1, (N_CHUNKS - 1) % NBUF).start()
    for j in range(NBUF - 1, -1, -1):
        g = N_CHUNKS - 1 - j
        out_copy(g, g % NBUF).wait()


def kernel(x, segment, strand, token_table, segment_table, strand_table):
    ctab, cidx = _prep(segment.astype(jnp.int32), strand.astype(jnp.int32),
                       segment_table, strand_table)
    xf = x.reshape(-1).astype(jnp.int32)
    cidxf = cidx.reshape(-1)

    mesh = plsc.VectorSubcoreMesh(core_axis_name="c", subcore_axis_name="s")
    run = functools.partial(
        pl.kernel,
        out_type=jax.ShapeDtypeStruct((N, D), jnp.float32),
        mesh=mesh,
        compiler_params=pltpu.CompilerParams(needs_layout_passes=False),
        scratch_types=[
            pltpu.VMEM_SHARED((2 * N_SEG, D), jnp.float32),
        ]
        + [pltpu.VMEM((K,), jnp.int32) for _ in range(NBUF)]
        + [pltpu.VMEM((K,), jnp.int32) for _ in range(NBUF)]
        + [pltpu.VMEM((K, D), jnp.float32) for _ in range(NBUF)]
        + [pltpu.SemaphoreType.DMA] * (4 * NBUF),
    )(_sc_body)
    out = run(xf, cidxf, token_table, ctab)
    return out.reshape(x.shape[0], x.shape[1], D)


# speedup vs baseline: 1.0063x; 1.0063x over previous
import functools

import jax
import jax.numpy as jnp
from jax import lax
from jax.experimental import pallas as pl
from jax.experimental.pallas import tpu as pltpu
from jax.experimental.pallas import tpu_sc as plsc

D = 128
N_SEG = 200
N = 4096 * 200
NC, NS, L = 2, 16, 16
NW = NC * NS
ROWS_PER_W = N // NW
K = 128
N_CHUNKS = ROWS_PER_W // K
NBUF = 5


def _prep_body(seg_ref, st_ref, segtab_ref, sttab_ref, ctab_ref, cidx_ref):
    ctab_ref[0:N_SEG, :] = segtab_ref[...] + sttab_ref[0:1, :]
    ctab_ref[N_SEG:2 * N_SEG, :] = segtab_ref[...] + sttab_ref[1:2, :]
    cidx_ref[...] = st_ref[...] * N_SEG + seg_ref[...]


def _prep(segment, strand, segment_table, strand_table):
    return pl.pallas_call(
        _prep_body,
        out_shape=(
            jax.ShapeDtypeStruct((2 * N_SEG, D), jnp.float32),
            jax.ShapeDtypeStruct(segment.shape, jnp.int32),
        ),
    )(segment, strand, segment_table, strand_table)


def _sc_body(x_hbm, cidx_hbm, tab_hbm, ctab_hbm, out_hbm, csh, *rest):
    wid = lax.axis_index("s") * NC + lax.axis_index("c")
    row0 = wid * ROWS_PER_W
    xb = rest[0:NBUF]
    cb = rest[NBUF:2 * NBUF]
    toks = rest[2 * NBUF:3 * NBUF]
    semi = rest[3 * NBUF:4 * NBUF]
    semg = rest[4 * NBUF:5 * NBUF]
    sema = rest[5 * NBUF:6 * NBUF]
    semo = rest[6 * NBUF:7 * NBUF]

    @pl.when(lax.axis_index("s") == 0)
    def _stage_ctab():
        pltpu.sync_copy(ctab_hbm, csh)

    plsc.subcore_barrier()

    def idx_copies(g, b):
        sl = pl.ds(row0 + g * K, K)
        return (pltpu.make_async_copy(x_hbm.at[sl], xb[b], semi[b]),
                pltpu.make_async_copy(cidx_hbm.at[sl], cb[b], semi[b]))

    def issue_idx(g, b):
        for cp in idx_copies(g, b):
            cp.start()

    def tok_copy(b):
        return pltpu.make_async_copy(tab_hbm.at[xb[b]], toks[b], semg[b])

    def add_copy(b):
        return pltpu.make_async_copy(csh.at[cb[b]], toks[b], sema[b])

    def out_copy(g, b):
        return pltpu.make_async_copy(
            toks[b], out_hbm.at[pl.ds(row0 + g * K, K)], semo[b])

    def launch_tok(g, b):
        for cp in idx_copies(g, b):
            cp.wait()
        tok_copy(b).start()

    for j in range(NBUF - 1):
        issue_idx(j, j)
    launch_tok(0, 0)
    launch_tok(1, 1)
    launch_tok(2, 2)

    def step(i, carry):
        for b in range(NBUF):
            g = NBUF * i + b

            def prev_out():
                add_copy((b - 1) % NBUF).wait()
                out_copy(g - 1, (b - 1) % NBUF).start()

            if b == 0:
                @pl.when(i >= 1)
                def _prev_out():
                    prev_out()
            else:
                prev_out()

            tok_copy(b).wait()
            add_copy(b).start(add=True)

            def next_tok():
                bn = (b + 3) % NBUF

                def drain():
                    out_copy(g - (NBUF - 3), bn).wait()

                if b < NBUF - 3:
                    @pl.when(i >= 1)
                    def _drain():
                        drain()
                else:
                    drain()
                launch_tok(g + 3, bn)

            if b <= NBUF - 4:
                next_tok()
            else:
                @pl.when(i <= N_CHUNKS // NBUF - 2)
                def _next_tok():
                    next_tok()

            if b <= NBUF - 5:
                issue_idx(g + 4, (b + 4) % NBUF)
            else:
                @pl.when(i <= N_CHUNKS // NBUF - 2)
                def _next_idx():
                    issue_idx(g + 4, (b + 4) % NBUF)
        return carry

    lax.fori_loop(0, N_CHUNKS // NBUF, step, 0)

    add_copy((N_CHUNKS - 1) % NBUF).wait()
    out_copy(N_CHUNKS - 1, (N_CHUNKS - 1) % NBUF).start()
    for j in range(NBUF - 1, -1, -1):
        g = N_CHUNKS - 1 - j
        out_copy(g, g % NBUF).wait()


def kernel(x, segment, strand, token_table, segment_table, strand_table):
    ctab, cidx = _prep(segment.astype(jnp.int32), strand.astype(jnp.int32),
                       segment_table, strand_table)
    xf = x.reshape(-1).astype(jnp.int32)
    cidxf = cidx.reshape(-1)

    mesh = plsc.VectorSubcoreMesh(core_axis_name="c", subcore_axis_name="s")
    run = functools.partial(
        pl.kernel,
        out_type=jax.ShapeDtypeStruct((N, D), jnp.float32),
        mesh=mesh,
        compiler_params=pltpu.CompilerParams(needs_layout_passes=False),
        scratch_types=[
            pltpu.VMEM_SHARED((2 * N_SEG, D), jnp.float32),
        ]
        + [pltpu.VMEM((K,), jnp.int32) for _ in range(NBUF)]
        + [pltpu.VMEM((K,), jnp.int32) for _ in range(NBUF)]
        + [pltpu.VMEM((K, D), jnp.float32) for _ in range(NBUF)]
        + [pltpu.SemaphoreType.DMA] * (4 * NBUF),
    )(_sc_body)
    out = run(xf, cidxf, token_table, ctab)
    return out.reshape(x.shape[0], x.shape[1], D)
